# baseline (speedup 1.0000x reference)
import jax
import jax.numpy as jnp
from jax import lax
from jax.experimental import pallas as pl
from jax.experimental.pallas import tpu as pltpu

_BLOCK = 32
_IN_F = 4096
_OUT_F = 4096
_GROUP = 128
_BLOCKS_PER_GROUP = _GROUP // _BLOCK
_N_GROUPS = _OUT_F // _GROUP
_QUAD = 4
_TOKEN_TILE = 512


def _spmm_body(rq_ref, cg_ref, x_ref, w_ref, bias_ref, o_ref, xb_ref, xq_ref):
    n_quads = w_ref.shape[0]
    xb_ref[...] = x_ref[...].astype(jnp.bfloat16)
    o_ref[...] = jnp.broadcast_to(bias_ref[...], o_ref.shape)

    def quad(q, carry):
        for j in range(_QUAD):
            cs = cg_ref[q * _QUAD + j]
            xq_ref[:, pl.ds(j * _GROUP, _GROUP)] = (
                xb_ref[:, pl.ds(cs * _GROUP, _GROUP)])
        contrib = jnp.dot(xq_ref[...], w_ref[q],
                          preferred_element_type=jnp.float32)
        rgq = rq_ref[q]
        o_ref[:, pl.ds(rgq * _GROUP, _GROUP)] += contrib
        return carry

    lax.fori_loop(0, n_quads, quad, 0, unroll=2)


@jax.jit
def kernel(x, weight_data, block_rows, block_cols, bias):
    n_tokens = x.shape[0]
    n_blocks = weight_data.shape[0]
    n_slots = n_blocks + (_QUAD - 1) * _N_GROUPS + (-n_blocks) % _QUAD
    n_quads = n_slots // _QUAD

    rg = (block_rows // _BLOCKS_PER_GROUP).astype(jnp.int32)
    ro = block_rows % _BLOCKS_PER_GROUP
    cg = (block_cols // _BLOCKS_PER_GROUP).astype(jnp.int32)
    co = block_cols % _BLOCKS_PER_GROUP

    wt = jnp.transpose(weight_data, (0, 2, 1))
    slots4 = jnp.arange(_BLOCKS_PER_GROUP, dtype=jnp.int32)
    oh_co = (co[:, None] == slots4).astype(jnp.float32)
    oh_ro = (ro[:, None] == slots4).astype(jnp.float32)
    w_tiles = (wt[:, None, :, None, :]
               * oh_co[:, :, None, None, None]
               * oh_ro[:, None, None, :, None])
    w_tiles = w_tiles.reshape(n_blocks, _GROUP, _GROUP)

    order = jnp.argsort(rg, stable=True)
    k_per_g = jnp.bincount(rg, length=_N_GROUPS)
    q_per_g = (k_per_g + _QUAD - 1) // _QUAD
    qcum = jnp.cumsum(q_per_g)
    kcum = jnp.cumsum(k_per_g)
    qstart = qcum - q_per_g
    kstart = kcum - k_per_g
    n_used_quads = qcum[-1]

    qids = jnp.arange(n_quads, dtype=jnp.int32)
    g_of_q = jnp.searchsorted(qcum, qids, side="right").astype(jnp.int32)
    g_of_q = jnp.minimum(g_of_q, _N_GROUPS - 1)
    rq = jnp.where(qids < n_used_quads, g_of_q, 0).astype(jnp.int32)

    sids = jnp.arange(n_slots, dtype=jnp.int32)
    s_q = sids // _QUAD
    s_g = g_of_q[s_q]
    pos_in_g = sids - _QUAD * qstart[s_g]
    valid = jnp.logical_and(s_q < n_used_quads, pos_in_g < k_per_g[s_g])
    bpos = jnp.clip(kstart[s_g] + pos_in_g, 0, n_blocks - 1)
    bid = order[bpos]

    w_slots = jnp.where(valid[:, None, None], w_tiles[bid], 0.0)
    w_quads = w_slots.astype(jnp.bfloat16).reshape(
        n_quads, _QUAD * _GROUP, _GROUP)
    cg_slots = jnp.where(valid, cg[bid], 0).astype(jnp.int32)

    bias2d = bias.reshape(1, _OUT_F)
    grid = (n_tokens // _TOKEN_TILE,)

    grid_spec = pltpu.PrefetchScalarGridSpec(
        num_scalar_prefetch=2,
        grid=grid,
        in_specs=[
            pl.BlockSpec((_TOKEN_TILE, _IN_F), lambda t, rq, cg: (t, 0)),
            pl.BlockSpec((n_quads, _QUAD * _GROUP, _GROUP),
                         lambda t, rq, cg: (0, 0, 0)),
            pl.BlockSpec((1, _OUT_F), lambda t, rq, cg: (0, 0)),
        ],
        out_specs=pl.BlockSpec((_TOKEN_TILE, _OUT_F),
                               lambda t, rq, cg: (t, 0)),
        scratch_shapes=[
            pltpu.VMEM((_TOKEN_TILE, _IN_F), jnp.bfloat16),
            pltpu.VMEM((_TOKEN_TILE, _QUAD * _GROUP), jnp.bfloat16),
        ],
    )

    return pl.pallas_call(
        _spmm_body,
        grid_spec=grid_spec,
        out_shape=jax.ShapeDtypeStruct((n_tokens, _OUT_F), jnp.float32),
        compiler_params=pltpu.CompilerParams(
            dimension_semantics=("arbitrary",),
        ),
    )(rq, cg_slots, x, w_quads, bias2d)


# speedup vs baseline: 1.1041x; 1.1041x over previous
import jax
import jax.numpy as jnp
from jax import lax
from jax.experimental import pallas as pl
from jax.experimental.pallas import tpu as pltpu

_BLOCK = 32
_IN_F = 4096
_OUT_F = 4096
_GROUP = 128
_BLOCKS_PER_GROUP = _GROUP // _BLOCK
_N_GROUPS = _OUT_F // _GROUP
_QUAD = 4
_TOKEN_TILE = 512


def _spmm_body(rq_ref, cg_ref, x_ref, w_ref, bias_ref, o_ref, xb_ref, xq_ref):
    n_quads = w_ref.shape[0]
    xb_ref[...] = x_ref[...].astype(jnp.bfloat16)
    o_ref[...] = jnp.broadcast_to(bias_ref[...], o_ref.shape)

    def quad(q, carry):
        buf = lax.rem(q, 2)
        for j in range(_QUAD):
            cs = cg_ref[q * _QUAD + j]
            xq_ref[buf, :, pl.ds(j * _GROUP, _GROUP)] = (
                xb_ref[:, pl.ds(cs * _GROUP, _GROUP)])
        contrib = jnp.dot(xq_ref[buf], w_ref[q],
                          preferred_element_type=jnp.float32)
        rgq = rq_ref[q]
        o_ref[:, pl.ds(rgq * _GROUP, _GROUP)] += contrib
        return carry

    lax.fori_loop(0, n_quads, quad, 0, unroll=2)


@jax.jit
def kernel(x, weight_data, block_rows, block_cols, bias):
    n_tokens = x.shape[0]
    n_blocks = weight_data.shape[0]
    n_slots = n_blocks + (_QUAD - 1) * _N_GROUPS + (-n_blocks) % _QUAD
    n_quads = n_slots // _QUAD

    rg = (block_rows // _BLOCKS_PER_GROUP).astype(jnp.int32)
    ro = block_rows % _BLOCKS_PER_GROUP
    cg = (block_cols // _BLOCKS_PER_GROUP).astype(jnp.int32)
    co = block_cols % _BLOCKS_PER_GROUP

    wt = jnp.transpose(weight_data, (0, 2, 1))
    slots4 = jnp.arange(_BLOCKS_PER_GROUP, dtype=jnp.int32)
    oh_co = (co[:, None] == slots4).astype(jnp.float32)
    oh_ro = (ro[:, None] == slots4).astype(jnp.float32)
    w_tiles = (wt[:, None, :, None, :]
               * oh_co[:, :, None, None, None]
               * oh_ro[:, None, None, :, None])
    w_tiles = w_tiles.reshape(n_blocks, _GROUP, _GROUP)

    order = jnp.argsort(rg, stable=True)
    k_per_g = jnp.bincount(rg, length=_N_GROUPS)
    q_per_g = (k_per_g + _QUAD - 1) // _QUAD
    qcum = jnp.cumsum(q_per_g)
    kcum = jnp.cumsum(k_per_g)
    qstart = qcum - q_per_g
    kstart = kcum - k_per_g
    n_used_quads = qcum[-1]

    qids = jnp.arange(n_quads, dtype=jnp.int32)
    g_of_q = jnp.searchsorted(qcum, qids, side="right").astype(jnp.int32)
    g_of_q = jnp.minimum(g_of_q, _N_GROUPS - 1)
    rq = jnp.where(qids < n_used_quads, g_of_q, 0).astype(jnp.int32)

    sids = jnp.arange(n_slots, dtype=jnp.int32)
    s_q = sids // _QUAD
    s_g = g_of_q[s_q]
    pos_in_g = sids - _QUAD * qstart[s_g]
    valid = jnp.logical_and(s_q < n_used_quads, pos_in_g < k_per_g[s_g])
    bpos = jnp.clip(kstart[s_g] + pos_in_g, 0, n_blocks - 1)
    bid = order[bpos]

    onehot = jnp.logical_and(
        bid[:, None] == jnp.arange(n_blocks, dtype=jnp.int32),
        valid[:, None]).astype(jnp.float32)
    w_slots = jnp.dot(onehot, w_tiles.reshape(n_blocks, _GROUP * _GROUP),
                      preferred_element_type=jnp.float32)
    w_quads = w_slots.astype(jnp.bfloat16).reshape(
        n_quads, _QUAD * _GROUP, _GROUP)
    cg_slots = jnp.where(valid, cg[bid], 0).astype(jnp.int32)

    bias2d = bias.reshape(1, _OUT_F)
    grid = (n_tokens // _TOKEN_TILE,)

    grid_spec = pltpu.PrefetchScalarGridSpec(
        num_scalar_prefetch=2,
        grid=grid,
        in_specs=[
            pl.BlockSpec((_TOKEN_TILE, _IN_F), lambda t, rq, cg: (t, 0)),
            pl.BlockSpec((n_quads, _QUAD * _GROUP, _GROUP),
                         lambda t, rq, cg: (0, 0, 0)),
            pl.BlockSpec((1, _OUT_F), lambda t, rq, cg: (0, 0)),
        ],
        out_specs=pl.BlockSpec((_TOKEN_TILE, _OUT_F),
                               lambda t, rq, cg: (t, 0)),
        scratch_shapes=[
            pltpu.VMEM((_TOKEN_TILE, _IN_F), jnp.bfloat16),
            pltpu.VMEM((2, _TOKEN_TILE, _QUAD * _GROUP), jnp.bfloat16),
        ],
    )

    return pl.pallas_call(
        _spmm_body,
        grid_spec=grid_spec,
        out_shape=jax.ShapeDtypeStruct((n_tokens, _OUT_F), jnp.float32),
        compiler_params=pltpu.CompilerParams(
            dimension_semantics=("arbitrary",),
        ),
    )(rq, cg_slots, x, w_quads, bias2d)
